# baseline (speedup 1.0000x reference)
import functools

import jax
import jax.numpy as jnp
from jax import lax
from jax.experimental import pallas as pl
from jax.experimental.pallas import tpu as pltpu
from jax.experimental.pallas import tpu_sc as plsc

B, H, W = 32, 512, 512
NUM_CLASSES = 4
R = 16
NCHUNK = H // R
NX = 2
NO = 3
UNROLL = 6
NMAIN = (NCHUNK // UNROLL) * UNROLL
LANES = 16

_mesh = plsc.VectorSubcoreMesh(core_axis_name="c", subcore_axis_name="s")


@functools.partial(
    pl.kernel,
    out_type=jax.ShapeDtypeStruct((B, NUM_CLASSES, H, W), jnp.float32),
    mesh=_mesh,
    compiler_params=pltpu.CompilerParams(use_tc_tiling_on_sc=True),
    scratch_types=(
        [pltpu.VMEM((R, W), jnp.float32) for _ in range(NX)]
        + [pltpu.VMEM((NUM_CLASSES, R, W), jnp.float32) for _ in range(NO)]
        + [pltpu.SemaphoreType.DMA for _ in range(NX + NO)]
    ),
)
def _onehot_sc(x_hbm, out_hbm, *scratch):
    x_bufs = scratch[:NX]
    o_bufs = scratch[NX:NX + NO]
    in_sems = scratch[NX + NO:NX + NO + NX]
    out_sems = scratch[NX + NO + NX:]
    num_cores = 2
    b = lax.axis_index("s") * num_cores + lax.axis_index("c")

    def in_src(j):
        return x_hbm.at[b, pl.ds(j * R, R), :]

    def out_dst(j):
        return out_hbm.at[b, :, pl.ds(j * R, R), :]

    def chunk(j, xb, ob, static_first=False):
        x_v, o_v = x_bufs[xb], o_bufs[ob]
        pltpu.make_async_copy(in_src(j), x_v, in_sems[xb]).wait()

        def drain_prev():
            pltpu.make_async_copy(o_v, out_dst(j - NO), out_sems[ob]).wait()

        if static_first:
            pass
        else:
            @pl.when(j >= NO)
            def _():
                drain_prev()

        @plsc.parallel_loop(0, W, step=LANES)
        def _vec(k):
            sl = pl.ds(k, LANES)
            one = jnp.ones((LANES,), jnp.float32)
            zero = jnp.zeros((LANES,), jnp.float32)
            for r in range(R):
                v = x_v[r, sl]
                s0 = jnp.where(v < 0.1, one, zero)
                s1 = jnp.where(v < 1.0, one, zero)
                s2 = jnp.where(v < 2.5, one, zero)
                o_v[0, r, sl] = s0
                o_v[1, r, sl] = s1 - s0
                o_v[2, r, sl] = s2 - s1
                o_v[3, r, sl] = one - s2

        pltpu.async_copy(o_v, out_dst(j), out_sems[ob])

        @pl.when(j + NX < NCHUNK)
        def _():
            pltpu.async_copy(in_src(j + NX), x_v, in_sems[xb])

    for t in range(NX):
        pltpu.async_copy(in_src(t), x_bufs[t], in_sems[t])

    def ring_body(i, carry):
        for t in range(UNROLL):
            chunk(i * UNROLL + t, t % NX, t % NO)
        return carry

    lax.fori_loop(0, NMAIN // UNROLL, ring_body, 0)

    for j in range(NMAIN, NCHUNK):
        chunk(j, j % NX, j % NO)

    for j in range(NCHUNK - NO, NCHUNK):
        pltpu.make_async_copy(o_bufs[j % NO], out_dst(j),
                              out_sems[j % NO]).wait()


def kernel(x):
    out = _onehot_sc(x.reshape(B, H, W))
    return out.reshape(B, 1, NUM_CLASSES, H, W)


# speedup vs baseline: 1.0314x; 1.0314x over previous
import functools

import jax
import jax.numpy as jnp
from jax import lax
from jax.experimental import pallas as pl
from jax.experimental.pallas import tpu as pltpu
from jax.experimental.pallas import tpu_sc as plsc

B, H, W = 32, 512, 512
NUM_CLASSES = 4
R = 16
NCHUNK = H // R
NBUF = 2
LANES = 16

_mesh = plsc.VectorSubcoreMesh(core_axis_name="c", subcore_axis_name="s")


@functools.partial(
    pl.kernel,
    out_type=jax.ShapeDtypeStruct((B, NUM_CLASSES, H, W), jnp.float32),
    mesh=_mesh,
    compiler_params=pltpu.CompilerParams(use_tc_tiling_on_sc=True),
    scratch_types=(
        [pltpu.VMEM((R, W), jnp.float32) for _ in range(NBUF)]
        + [pltpu.VMEM((NUM_CLASSES, R, W), jnp.float32) for _ in range(NBUF)]
        + [pltpu.SemaphoreType.DMA for _ in range(2 * NBUF)]
    ),
)
def _onehot_sc(x_hbm, out_hbm, *scratch):
    x_bufs = scratch[:NBUF]
    o_bufs = scratch[NBUF:2 * NBUF]
    in_sems = scratch[2 * NBUF:3 * NBUF]
    out_sems = scratch[3 * NBUF:4 * NBUF]
    num_cores = 2
    b = lax.axis_index("s") * num_cores + lax.axis_index("c")

    def in_src(j):
        return x_hbm.at[b, pl.ds(j * R, R), :]

    def out_dst(j):
        return out_hbm.at[b, :, pl.ds(j * R, R), :]

    for t in range(NBUF):
        pltpu.async_copy(in_src(t), x_bufs[t], in_sems[t])

    def ring_body(i, carry):
        for t in range(NBUF):
            j = i * NBUF + t
            x_v, o_v = x_bufs[t], o_bufs[t]
            pltpu.make_async_copy(in_src(j), x_v, in_sems[t]).wait()

            @pl.when(j >= NBUF)
            def _():
                pltpu.make_async_copy(
                    o_v, out_dst(j - NBUF), out_sems[t]).wait()

            @plsc.parallel_loop(0, W, step=LANES)
            def _vec(k):
                sl = pl.ds(k, LANES)
                one = jnp.ones((LANES,), jnp.float32)
                zero = jnp.zeros((LANES,), jnp.float32)
                for r in range(R):
                    v = x_v[r, sl]
                    s0 = jnp.where(v < 0.1, one, zero)
                    s1 = jnp.where(v < 1.0, one, zero)
                    s2 = jnp.where(v < 2.5, one, zero)
                    o_v[0, r, sl] = s0
                    o_v[1, r, sl] = s1 - s0
                    o_v[2, r, sl] = s2 - s1
                    o_v[3, r, sl] = one - s2

            pltpu.async_copy(o_v, out_dst(j), out_sems[t])

            @pl.when(j + NBUF < NCHUNK)
            def _():
                pltpu.async_copy(in_src(j + NBUF), x_v, in_sems[t])
        return carry

    lax.fori_loop(0, NCHUNK // NBUF, ring_body, 0)

    for t in range(NBUF):
        j = NCHUNK - NBUF + t
        pltpu.make_async_copy(o_bufs[t], out_dst(j), out_sems[t]).wait()


def kernel(x):
    out = _onehot_sc(x.reshape(B, H, W))
    return out.reshape(B, 1, NUM_CLASSES, H, W)
